# baseline (speedup 1.0000x reference)
import jax
import jax.numpy as jnp
from jax import lax
from jax.experimental import pallas as pl
from jax.experimental.pallas import tpu as pltpu
from jax.experimental.pallas import tpu_sc as plsc

_B = 4096
_N_NUM = 10
_N_CAT = 26
_VOCAB = 100000
_D = 32
_SEQ = 1 + _N_NUM + _N_CAT

_NC = 2
_NS = 16
_NW = _NC * _NS

_CAT_ROWS = _N_CAT * _D
_AUX_ROWS = (1 + _N_NUM) * _D
_CAT_PER_W = _CAT_ROWS // _NW
_AUX_PER_W = _AUX_ROWS // _NW
_LANES = 16
_HALF0 = 50048
_HALF1 = _VOCAB - _HALF0
_GRP = _B // _LANES


def _tokenize_body(tables_hbm, cat_hbm, num_hbm, nk_hbm, bias_hbm,
                   out_hbm,
                   idx2_v, h0_v, h1_v, v0_v, v1_v, nk_v, bias_v,
                   sh0, sh1, so0, so1):
    wid = lax.axis_index("s") * _NC + lax.axis_index("c")

    pltpu.sync_copy(nk_hbm, nk_v)
    pltpu.sync_copy(bias_hbm, bias_v)
    hbufs = (h0_v, h1_v)
    hsems = (sh0, sh1)
    vbufs = (v0_v, v1_v)
    osems = (so0, so1)

    r0 = wid * _CAT_PER_W
    j0 = r0 // _D
    j1 = jnp.minimum(j0 + 1, _N_CAT - 1)
    pltpu.sync_copy(cat_hbm.at[j0], idx2_v.at[pl.ds(0, _B)])
    pltpu.sync_copy(cat_hbm.at[j1], idx2_v.at[pl.ds(_B, _B)])

    copies = [None, None]
    ocopies = [None, None]

    def fire(k, h):
        off, size = (0, _HALF0) if h == 0 else (_HALF0, _HALF1)
        copies[h] = pltpu.async_copy(
            tables_hbm.at[r0 + k].at[pl.ds(off, size)],
            hbufs[h], hsems[h])

    def gather_half(k, h):
        copies[h].wait()
        buf = hbufs[h]
        val = vbufs[k % 2]
        base, size = (0, _HALF0) if h == 0 else (_HALF0, _HALF1)
        r = r0 + k
        sel_off = ((r // _D) - j0) * _B
        bval = bias_v[pl.ds(_AUX_ROWS + r, _LANES)][0]

        def body(i, c):
            sl = pl.ds(sel_off + i * _LANES, _LANES)
            loc = idx2_v[sl] - base
            locc = jnp.minimum(jnp.maximum(loc, 0), size - 1)
            g = plsc.load_gather(buf, [locc]) + bval
            osl = pl.ds(i * _LANES, _LANES)
            if h == 0:
                val[osl] = jnp.where(loc < _HALF0, g, jnp.float32(0.0))
            else:
                val[osl] = jnp.where(loc >= 0, g, val[osl])
            return c
        lax.fori_loop(0, _GRP, body, 0)

    fire(0, 0)
    for k in range(_CAT_PER_W):
        fire(k, 1)
        if k >= 2:
            ocopies[k % 2].wait()
        gather_half(k, 0)
        if k + 1 < _CAT_PER_W:
            fire(k + 1, 0)
        gather_half(k, 1)
        ocopies[k % 2] = pltpu.async_copy(
            vbufs[k % 2], out_hbm.at[_AUX_ROWS + r0 + k], osems[k % 2])
    ocopies[0].wait()
    ocopies[1].wait()

    ncopies = [None, None]
    nocopies = [None, None]

    def nfire(m):
        s = (wid * _AUX_PER_W + m) // _D
        f_safe = jnp.maximum(s - 1, 0)
        ncopies[m % 2] = pltpu.async_copy(
            num_hbm.at[f_safe], hbufs[m % 2].at[pl.ds(0, _B)], hsems[m % 2])

    nfire(0)
    for m in range(_AUX_PER_W):
        a = wid * _AUX_PER_W + m
        if m + 1 < _AUX_PER_W:
            nfire(m + 1)
        s = a // _D
        d = a % _D
        f_safe = jnp.maximum(s - 1, 0)
        nk_raw = nk_v[pl.ds(f_safe * _D + d, _LANES)][0]
        nkval = jnp.where(s == 0, jnp.float32(0.0), nk_raw)
        bval = bias_v[pl.ds(s * _D + d, _LANES)][0]
        ncopies[m % 2].wait()
        if m >= 2:
            nocopies[m % 2].wait()
        src = hbufs[m % 2]
        val = vbufs[m % 2]

        def fma(i, c):
            sl = pl.ds(i * _LANES, _LANES)
            val[sl] = src[sl] * nkval + bval
            return c
        lax.fori_loop(0, _GRP, fma, 0)
        nocopies[m % 2] = pltpu.async_copy(
            val, out_hbm.at[a], osems[m % 2])
    nocopies[0].wait()
    nocopies[1].wait()


@jax.jit
def kernel(numerical, categorical, numerical_kernel, tables, bias_kernel):
    tables_t = tables.transpose(0, 2, 1).reshape(_CAT_ROWS, _VOCAB)
    cat_t = categorical.T
    num_t = numerical.T
    nk_flat = jnp.pad(numerical_kernel.reshape(_N_NUM * _D), (0, _LANES))
    bias_flat = jnp.pad(bias_kernel.reshape(_SEQ * _D), (0, _LANES))

    mesh = plsc.VectorSubcoreMesh(core_axis_name="c", subcore_axis_name="s")
    run = pl.kernel(
        _tokenize_body,
        out_type=jax.ShapeDtypeStruct((_SEQ * _D, _B), jnp.float32),
        mesh=mesh,
        compiler_params=pltpu.CompilerParams(needs_layout_passes=False),
        scratch_types=[
            pltpu.VMEM((2 * _B,), jnp.int32),
            pltpu.VMEM((_HALF0,), jnp.float32),
            pltpu.VMEM((_HALF1,), jnp.float32),
            pltpu.VMEM((_B,), jnp.float32),
            pltpu.VMEM((_B,), jnp.float32),
            pltpu.VMEM((_N_NUM * _D + _LANES,), jnp.float32),
            pltpu.VMEM((_SEQ * _D + _LANES,), jnp.float32),
            pltpu.SemaphoreType.DMA,
            pltpu.SemaphoreType.DMA,
            pltpu.SemaphoreType.DMA,
            pltpu.SemaphoreType.DMA,
        ],
    )
    out_t = run(tables_t, cat_t, num_t, nk_flat, bias_flat)
    return out_t.reshape(_SEQ, _D, _B).transpose(2, 0, 1)


# speedup vs baseline: 1.2826x; 1.2826x over previous
import jax
import jax.numpy as jnp
from jax import lax
from jax.experimental import pallas as pl
from jax.experimental.pallas import tpu as pltpu
from jax.experimental.pallas import tpu_sc as plsc

_B = 4096
_N_NUM = 10
_N_CAT = 26
_VOCAB = 100000
_D = 32
_SEQ = 1 + _N_NUM + _N_CAT

_NC = 2
_NS = 16

_CAT_ROWS = _N_CAT * _D
_AUX_ROWS = (1 + _N_NUM) * _D
_AUX_PER_W = 59
_LANES = 16
_HALF0 = 50048
_HALF1 = _VOCAB - _HALF0
_GRP = _B // _LANES


def _tokenize_body(tables_hbm, cat_hbm, num_hbm, nk_hbm, bias_hbm,
                   out_hbm,
                   idx_v, h0_v, h1_v, v0_v, v1_v, nk_v, bias_v,
                   sh0, sh1, so0, so1):
    wid = lax.axis_index("s") * _NC + lax.axis_index("c")

    pltpu.sync_copy(nk_hbm, nk_v)
    pltpu.sync_copy(bias_hbm, bias_v)
    hbufs = (h0_v, h1_v)
    hsems = (sh0, sh1)
    vbufs = (v0_v, v1_v)
    osems = (so0, so1)

    @pl.when(wid < _N_CAT)
    def _cat_worker():
        j = wid
        pltpu.sync_copy(cat_hbm.at[j], idx_v)
        copies = [None, None]
        ocopies = [None, None]

        def fire(d, h):
            off, size = (0, _HALF0) if h == 0 else (_HALF0, _HALF1)
            copies[h] = pltpu.async_copy(
                tables_hbm.at[j * _D + d].at[pl.ds(off, size)],
                hbufs[h], hsems[h])

        def gather_half(d, h):
            copies[h].wait()
            buf = hbufs[h]
            val = vbufs[d % 2]
            base, size = (0, _HALF0) if h == 0 else (_HALF0, _HALF1)
            bval = bias_v[pl.ds((_N_NUM + 1 + j) * _D + d, _LANES)][0]

            def body(i, c):
                sl = pl.ds(i * _LANES, _LANES)
                loc = idx_v[sl] - base
                locc = jnp.minimum(jnp.maximum(loc, 0), size - 1)
                g = plsc.load_gather(buf, [locc]) + bval
                if h == 0:
                    val[sl] = jnp.where(loc < _HALF0, g, jnp.float32(0.0))
                else:
                    val[sl] = jnp.where(loc >= 0, g, val[sl])
                return c
            lax.fori_loop(0, _GRP, body, 0)

        fire(0, 0)
        for d in range(_D):
            fire(d, 1)
            if d >= 2:
                ocopies[d % 2].wait()
            gather_half(d, 0)
            if d + 1 < _D:
                fire(d + 1, 0)
            gather_half(d, 1)
            ocopies[d % 2] = pltpu.async_copy(
                vbufs[d % 2], out_hbm.at[(_N_NUM + 1 + j) * _D + d],
                osems[d % 2])
        ocopies[0].wait()
        ocopies[1].wait()

    @pl.when(wid >= _N_CAT)
    def _aux_worker():
        aw = wid - _N_CAT
        ncopies = [None, None]
        nocopies = [None, None]

        def run_idx(m):
            return jnp.minimum(aw * _AUX_PER_W + m, _AUX_ROWS - 1)

        def nfire(m):
            s = run_idx(m) // _D
            f_safe = jnp.maximum(s - 1, 0)
            ncopies[m % 2] = pltpu.async_copy(
                num_hbm.at[f_safe],
                hbufs[m % 2].at[pl.ds(0, _B)], hsems[m % 2])

        nfire(0)
        for m in range(_AUX_PER_W):
            a = run_idx(m)
            if m + 1 < _AUX_PER_W:
                nfire(m + 1)
            s = a // _D
            d = a % _D
            f_safe = jnp.maximum(s - 1, 0)
            nk_raw = nk_v[pl.ds(f_safe * _D + d, _LANES)][0]
            nkval = jnp.where(s == 0, jnp.float32(0.0), nk_raw)
            bval = bias_v[pl.ds(s * _D + d, _LANES)][0]
            ncopies[m % 2].wait()
            if m >= 2:
                nocopies[m % 2].wait()
            src = hbufs[m % 2]
            val = vbufs[m % 2]

            def fma(i, c):
                sl = pl.ds(i * _LANES, _LANES)
                val[sl] = src[sl] * nkval + bval
                return c
            lax.fori_loop(0, _GRP, fma, 0)
            nocopies[m % 2] = pltpu.async_copy(
                val, out_hbm.at[a], osems[m % 2])
        nocopies[0].wait()
        nocopies[1].wait()


@jax.jit
def kernel(numerical, categorical, numerical_kernel, tables, bias_kernel):
    tables_t = tables.transpose(0, 2, 1).reshape(_CAT_ROWS, _VOCAB)
    cat_t = categorical.T
    num_t = numerical.T
    nk_flat = jnp.pad(numerical_kernel.reshape(_N_NUM * _D), (0, _LANES))
    bias_flat = jnp.pad(bias_kernel.reshape(_SEQ * _D), (0, _LANES))

    mesh = plsc.VectorSubcoreMesh(core_axis_name="c", subcore_axis_name="s")
    run = pl.kernel(
        _tokenize_body,
        out_type=jax.ShapeDtypeStruct((_SEQ * _D, _B), jnp.float32),
        mesh=mesh,
        compiler_params=pltpu.CompilerParams(needs_layout_passes=False),
        scratch_types=[
            pltpu.VMEM((_B,), jnp.int32),
            pltpu.VMEM((_HALF0,), jnp.float32),
            pltpu.VMEM((_HALF1,), jnp.float32),
            pltpu.VMEM((_B,), jnp.float32),
            pltpu.VMEM((_B,), jnp.float32),
            pltpu.VMEM((_N_NUM * _D + _LANES,), jnp.float32),
            pltpu.VMEM((_SEQ * _D + _LANES,), jnp.float32),
            pltpu.SemaphoreType.DMA,
            pltpu.SemaphoreType.DMA,
            pltpu.SemaphoreType.DMA,
            pltpu.SemaphoreType.DMA,
        ],
    )
    out_t = run(tables_t, cat_t, num_t, nk_flat, bias_flat)
    return out_t.reshape(_SEQ, _D, _B).transpose(2, 0, 1)
